# baseline (speedup 1.0000x reference)
import functools

import jax
import jax.numpy as jnp
from jax import lax
from jax.experimental import pallas as pl
from jax.experimental.pallas import tpu as pltpu
from jax.experimental.pallas import tpu_sc as plsc

N_FACTORS = 32
LANES = 16
W = 1024
RING = 8


def _pad128(n):
    return -(-n // 128) * 128


@functools.lru_cache(maxsize=None)
def _build_sc(batch: int, n_users: int, n_movies: int):
    try:
        info = plsc.get_sparse_core_info()
        num_cores, num_subcores = info.num_cores, info.num_subcores
    except Exception:
        num_cores, num_subcores = 2, 16
    num_workers = num_cores * num_subcores
    n_vecs = batch // LANES

    u_pad, m_pad = _pad128(n_users), _pad128(n_movies)
    u_windows = -(-u_pad // W)
    m_windows = -(-m_pad // W)
    u_per_w = -(-u_windows // num_workers)
    m_per_w = -(-m_windows // num_workers)

    mesh = plsc.VectorSubcoreMesh(core_axis_name="c", subcore_axis_name="s")

    @functools.partial(
        pl.kernel,
        mesh=mesh,
        out_type=(jax.ShapeDtypeStruct((batch * N_FACTORS,), jnp.float32),
                  jax.ShapeDtypeStruct((batch * N_FACTORS,), jnp.float32)),
        scratch_types=[
            pltpu.VMEM((batch,), jnp.int32),
            pltpu.VMEM((batch,), jnp.int32),
            pltpu.VMEM((batch,), jnp.int32),
            pltpu.VMEM((N_FACTORS, W), jnp.float32),
            pltpu.VMEM((N_FACTORS, W), jnp.float32),
            pltpu.VMEM((RING * N_FACTORS,), jnp.float32),
            pltpu.SemaphoreType.DMA,
            pltpu.SemaphoreType.DMA,
            pltpu.SemaphoreType.DMA,
        ],
        compiler_params=pltpu.CompilerParams(needs_layout_passes=False),
    )
    def sc_kernel(users, movies, ut, mt, u_flat, m_flat,
                  raw, memb_b, memb_v, win0, win1, stag, sem0, sem1, semr):
        wid = lax.axis_index("s") * num_cores + lax.axis_index("c")
        row0 = lax.iota(jnp.int32, 16)
        row1 = row0 + LANES
        lane = row0

        def phase(src_idx, tbl, flat, per_worker, windows_total, last_start):
            first = wid * per_worker
            nwin = jnp.maximum(
                jnp.minimum(per_worker, windows_total - first), 0)
            npairs = (nwin + 1) // 2

            def start_of(n):
                c = jnp.minimum((first + n) * W, last_start)
                return pl.multiple_of(c, 128)

            lo = start_of(0)
            hi = start_of(nwin - 1) + W

            pltpu.async_copy(tbl.at[:, pl.ds(lo, W)], win0, sem0)
            pltpu.sync_copy(src_idx, raw)

            def compact(g, nm):
                v = raw[pl.ds(g * LANES, LANES)]
                m = jnp.logical_and(v >= lo, v < hi)
                pos = plsc.cumsum(m.astype(jnp.int32))
                dst = nm + pos - 1
                plsc.store_scatter(memb_b, [dst], g * LANES + lane, mask=m)
                plsc.store_scatter(memb_v, [dst], v, mask=m)
                return nm + pos[15]

            nm = lax.fori_loop(0, n_vecs, compact, 0)
            n_mvecs = (nm + LANES - 1) // LANES

            def extract(win, c0, hit):
                def group(g, hit):
                    bs = memb_b[pl.ds(g * LANES, LANES)]
                    vs = memb_v[pl.ds(g * LANES, LANES)]
                    col = vs - c0
                    valid = (g * LANES + lane) < nm
                    m = (col >= 0) & (col < W) & valid

                    mi = m.astype(jnp.int32)

                    def lanes_body():
                        h = hit
                        for l in range(LANES):
                            ml = mi[l]
                            slot = h % RING

                            @pl.when(ml == 1)
                            def _(l=l, h=h, slot=slot):
                                @pl.when(h >= RING)
                                def _():
                                    pltpu.make_async_copy(
                                        flat.at[pl.ds(0, N_FACTORS)],
                                        stag.at[pl.ds(slot * N_FACTORS,
                                                      N_FACTORS)],
                                        semr).wait()
                                safe = jnp.clip(col[l], 0, W - 1)
                                cs = jnp.broadcast_to(safe, (LANES,))
                                g0 = plsc.load_gather(win, [row0, cs])
                                g1 = plsc.load_gather(win, [row1, cs])
                                s0 = slot * N_FACTORS
                                stag[pl.ds(s0, LANES)] = g0
                                stag[pl.ds(s0 + LANES, LANES)] = g1
                                pltpu.async_copy(
                                    stag.at[pl.ds(s0, N_FACTORS)],
                                    flat.at[pl.ds(bs[l] * N_FACTORS,
                                                  N_FACTORS)],
                                    semr)

                            h = h + ml
                        return h

                    any_hit = jnp.sum(mi) > 0
                    return lax.cond(any_hit, lanes_body, lambda: hit)

                return lax.fori_loop(0, n_mvecs, group, hit)

            def pair(p, hit):
                n0 = 2 * p
                c0, c1, c2 = start_of(n0), start_of(n0 + 1), start_of(n0 + 2)
                pltpu.async_copy(tbl.at[:, pl.ds(c1, W)], win1, sem1)
                pltpu.make_async_copy(tbl.at[:, pl.ds(c1, W)], win0,
                                      sem0).wait()
                hit = extract(win0, c0, hit)
                pltpu.async_copy(tbl.at[:, pl.ds(c2, W)], win0, sem0)
                pltpu.make_async_copy(tbl.at[:, pl.ds(c2, W)], win1,
                                      sem1).wait()
                hit = extract(win1, c1, hit)
                return hit

            hit = lax.fori_loop(0, npairs, pair, 0)
            pltpu.make_async_copy(tbl.at[:, pl.ds(0, W)], win0, sem0).wait()

            def drain(i, carry):
                pltpu.make_async_copy(
                    flat.at[pl.ds(0, N_FACTORS)],
                    stag.at[pl.ds(0, N_FACTORS)], semr).wait()
                return carry

            lax.fori_loop(0, jnp.minimum(hit, RING), drain, 0)

        phase(users, ut, u_flat, u_per_w, u_windows, u_pad - W)
        phase(movies, mt, m_flat, m_per_w, m_windows, m_pad - W)

    return sc_kernel


def _dot_body(u_ref, m_ref, o_ref):
    u = u_ref[...].reshape(256, 128)
    m = m_ref[...].reshape(256, 128)
    p = u * m
    grp = lax.broadcasted_iota(jnp.int32, (128, 4), 0) // N_FACTORS
    col = lax.broadcasted_iota(jnp.int32, (128, 4), 1)
    sel = (grp == col).astype(jnp.float32)
    o_ref[...] = jnp.dot(p, sel, preferred_element_type=jnp.float32)


@functools.lru_cache(maxsize=None)
def _build_dot(batch: int):
    n_blocks = batch * N_FACTORS // 32768
    return pl.pallas_call(
        _dot_body,
        grid=(n_blocks,),
        in_specs=[pl.BlockSpec((32768,), lambda i: (i,)),
                  pl.BlockSpec((32768,), lambda i: (i,))],
        out_specs=pl.BlockSpec((256, 4), lambda i: (i, 0)),
        out_shape=jax.ShapeDtypeStruct((batch // 4, 4), jnp.float32),
    )


def kernel(users, movies, user_table, movie_table):
    batch = users.shape[0]
    sc = _build_sc(batch, user_table.shape[0], movie_table.shape[0])
    u_flat, m_flat = sc(users, movies, user_table.T, movie_table.T)
    preds4 = _build_dot(batch)(u_flat, m_flat)
    return preds4.reshape(batch)


# speedup vs baseline: 1.0324x; 1.0324x over previous
import functools

import jax
import jax.numpy as jnp
from jax import lax
from jax.experimental import pallas as pl
from jax.experimental.pallas import tpu as pltpu
from jax.experimental.pallas import tpu_sc as plsc

N_FACTORS = 32
LANES = 16
W = 1024
RING = 8


def _pad128(n):
    return -(-n // 128) * 128


@functools.lru_cache(maxsize=None)
def _build_sc(batch: int, n_users: int, n_movies: int):
    try:
        info = plsc.get_sparse_core_info()
        num_cores, num_subcores = info.num_cores, info.num_subcores
    except Exception:
        num_cores, num_subcores = 2, 16
    num_workers = num_cores * num_subcores
    n_vecs = batch // LANES

    u_pad, m_pad = _pad128(n_users), _pad128(n_movies)
    u_windows = -(-u_pad // W)
    m_windows = -(-m_pad // W)
    u_per_w = -(-u_windows // num_workers)
    m_per_w = -(-m_windows // num_workers)

    mesh = plsc.VectorSubcoreMesh(core_axis_name="c", subcore_axis_name="s")

    @functools.partial(
        pl.kernel,
        mesh=mesh,
        out_type=(jax.ShapeDtypeStruct((batch * N_FACTORS,), jnp.float32),
                  jax.ShapeDtypeStruct((batch * N_FACTORS,), jnp.float32)),
        scratch_types=[
            pltpu.VMEM((batch,), jnp.int32),
            pltpu.VMEM((batch,), jnp.int32),
            pltpu.VMEM((batch,), jnp.int32),
            pltpu.VMEM((N_FACTORS, W), jnp.float32),
            pltpu.VMEM((N_FACTORS, W), jnp.float32),
            pltpu.VMEM((RING * N_FACTORS,), jnp.float32),
            pltpu.SemaphoreType.DMA,
            pltpu.SemaphoreType.DMA,
            pltpu.SemaphoreType.DMA,
        ],
        compiler_params=pltpu.CompilerParams(needs_layout_passes=False),
    )
    def sc_kernel(users, movies, ut, mt, u_flat, m_flat,
                  raw, memb_b, memb_v, win0, win1, stag, sem0, sem1, semr):
        wid = lax.axis_index("s") * num_cores + lax.axis_index("c")
        row0 = lax.iota(jnp.int32, 16)
        row1 = row0 + LANES
        lane = row0

        def phase(src_idx, tbl, flat, per_worker, windows_total, last_start):
            first = wid * per_worker
            nwin = jnp.maximum(
                jnp.minimum(per_worker, windows_total - first), 0)
            npairs = (nwin + 1) // 2

            def start_of(n):
                c = jnp.minimum((first + n) * W, last_start)
                return pl.multiple_of(c, 128)

            lo = start_of(0)
            hi = start_of(nwin - 1) + W

            pltpu.async_copy(tbl.at[:, pl.ds(lo, W)], win0, sem0)
            pltpu.sync_copy(src_idx, raw)

            def compact(q, nm):
                vs, ms, poss = [], [], []
                for t in range(4):
                    v = raw[pl.ds((4 * q + t) * LANES, LANES)]
                    m = jnp.logical_and(v >= lo, v < hi)
                    vs.append(v)
                    ms.append(m)
                    poss.append(plsc.cumsum(m.astype(jnp.int32)))
                for t in range(4):
                    dst = nm + poss[t] - 1
                    plsc.store_scatter(memb_b, [dst],
                                       (4 * q + t) * LANES + lane,
                                       mask=ms[t])
                    plsc.store_scatter(memb_v, [dst], vs[t], mask=ms[t])
                    nm = nm + poss[t][15]
                return nm

            nm = lax.fori_loop(0, n_vecs // 4, compact, 0)
            n_mvecs = (nm + LANES - 1) // LANES

            def extract(win, c0, hit):
                def group(g, hit):
                    bs = memb_b[pl.ds(g * LANES, LANES)]
                    vs = memb_v[pl.ds(g * LANES, LANES)]
                    col = vs - c0
                    valid = (g * LANES + lane) < nm
                    m = (col >= 0) & (col < W) & valid

                    mi = m.astype(jnp.int32)

                    def lanes_body():
                        h = hit
                        for l in range(LANES):
                            ml = mi[l]
                            slot = h % RING

                            @pl.when(ml == 1)
                            def _(l=l, h=h, slot=slot):
                                @pl.when(h >= RING)
                                def _():
                                    pltpu.make_async_copy(
                                        flat.at[pl.ds(0, N_FACTORS)],
                                        stag.at[pl.ds(slot * N_FACTORS,
                                                      N_FACTORS)],
                                        semr).wait()
                                safe = jnp.clip(col[l], 0, W - 1)
                                cs = jnp.broadcast_to(safe, (LANES,))
                                g0 = plsc.load_gather(win, [row0, cs])
                                g1 = plsc.load_gather(win, [row1, cs])
                                s0 = slot * N_FACTORS
                                stag[pl.ds(s0, LANES)] = g0
                                stag[pl.ds(s0 + LANES, LANES)] = g1
                                pltpu.async_copy(
                                    stag.at[pl.ds(s0, N_FACTORS)],
                                    flat.at[pl.ds(bs[l] * N_FACTORS,
                                                  N_FACTORS)],
                                    semr)

                            h = h + ml
                        return h

                    any_hit = jnp.any(m)
                    return lax.cond(any_hit, lanes_body, lambda: hit)

                return lax.fori_loop(0, n_mvecs, group, hit)

            def pair(p, hit):
                n0 = 2 * p
                c0, c1, c2 = start_of(n0), start_of(n0 + 1), start_of(n0 + 2)
                pltpu.async_copy(tbl.at[:, pl.ds(c1, W)], win1, sem1)
                pltpu.make_async_copy(tbl.at[:, pl.ds(c1, W)], win0,
                                      sem0).wait()
                hit = extract(win0, c0, hit)
                pltpu.async_copy(tbl.at[:, pl.ds(c2, W)], win0, sem0)
                pltpu.make_async_copy(tbl.at[:, pl.ds(c2, W)], win1,
                                      sem1).wait()
                hit = extract(win1, c1, hit)
                return hit

            hit = lax.fori_loop(0, npairs, pair, 0)
            pltpu.make_async_copy(tbl.at[:, pl.ds(0, W)], win0, sem0).wait()

            def drain(i, carry):
                pltpu.make_async_copy(
                    flat.at[pl.ds(0, N_FACTORS)],
                    stag.at[pl.ds(0, N_FACTORS)], semr).wait()
                return carry

            lax.fori_loop(0, jnp.minimum(hit, RING), drain, 0)

        phase(users, ut, u_flat, u_per_w, u_windows, u_pad - W)
        phase(movies, mt, m_flat, m_per_w, m_windows, m_pad - W)

    return sc_kernel


def _dot_body(u_ref, m_ref, o_ref):
    u = u_ref[...].reshape(256, 128)
    m = m_ref[...].reshape(256, 128)
    p = u * m
    grp = lax.broadcasted_iota(jnp.int32, (128, 4), 0) // N_FACTORS
    col = lax.broadcasted_iota(jnp.int32, (128, 4), 1)
    sel = (grp == col).astype(jnp.float32)
    o_ref[...] = jnp.dot(p, sel, preferred_element_type=jnp.float32)


@functools.lru_cache(maxsize=None)
def _build_dot(batch: int):
    n_blocks = batch * N_FACTORS // 32768
    return pl.pallas_call(
        _dot_body,
        grid=(n_blocks,),
        in_specs=[pl.BlockSpec((32768,), lambda i: (i,)),
                  pl.BlockSpec((32768,), lambda i: (i,))],
        out_specs=pl.BlockSpec((256, 4), lambda i: (i, 0)),
        out_shape=jax.ShapeDtypeStruct((batch // 4, 4), jnp.float32),
    )


def kernel(users, movies, user_table, movie_table):
    batch = users.shape[0]
    sc = _build_sc(batch, user_table.shape[0], movie_table.shape[0])
    u_flat, m_flat = sc(users, movies, user_table.T, movie_table.T)
    preds4 = _build_dot(batch)(u_flat, m_flat)
    return preds4.reshape(batch)
